# baseline (speedup 1.0000x reference)
import functools

import jax
import jax.numpy as jnp
from jax import lax
from jax.experimental import pallas as pl
from jax.experimental.pallas import tpu as pltpu
from jax.experimental.pallas import tpu_sc as plsc

_CHUNK = 128


@functools.lru_cache(maxsize=None)
def _make_gather(V, D, N):
    info = plsc.get_sparse_core_info()
    NC, NS = info.num_cores, info.num_subcores
    NW = NC * NS
    assert N % (NW * _CHUNK) == 0
    b_per_w = N // NW
    n_chunks = b_per_w // _CHUNK

    mesh = plsc.VectorSubcoreMesh(core_axis_name="c", subcore_axis_name="s")

    @functools.partial(
        pl.kernel,
        mesh=mesh,
        compiler_params=pltpu.CompilerParams(use_tc_tiling_on_sc=False),
        out_type=jax.ShapeDtypeStruct((N, D), jnp.float32),
        scratch_types=[
            pltpu.VMEM((n_chunks, _CHUNK), jnp.int32),
            pltpu.VMEM((_CHUNK, D), jnp.float32),
            pltpu.SemaphoreType.DMA,
        ],
    )
    def k(idx_hbm, table_hbm, out_hbm, idx_v, rows_v, gsem):
        wid = lax.axis_index("s") * NC + lax.axis_index("c")
        base = wid * b_per_w
        pltpu.sync_copy(idx_hbm.at[wid], idx_v)

        def body(j, carry):
            pltpu.async_copy(table_hbm.at[idx_v.at[j]], rows_v, gsem).wait()
            pltpu.sync_copy(rows_v, out_hbm.at[pl.ds(base + j * _CHUNK, _CHUNK)])
            return carry

        lax.fori_loop(0, n_chunks, body, 0)

    def run(idx2d, table):
        return k(idx2d, table)

    return run


def kernel(tokens, table):
    B, L = tokens.shape
    V, D = table.shape
    N = B * L
    info = plsc.get_sparse_core_info()
    NW = info.num_cores * info.num_subcores
    idx2d = tokens.astype(jnp.int32).reshape(NW, N // (NW * _CHUNK), _CHUNK)
    out = _make_gather(V, D, N)(idx2d, table)
    return out.reshape(B, L, D)


# speedup vs baseline: 1.0946x; 1.0946x over previous
import functools

import jax
import jax.numpy as jnp
from jax import lax
from jax.experimental import pallas as pl
from jax.experimental.pallas import tpu as pltpu
from jax.experimental.pallas import tpu_sc as plsc

_CHUNK = 128


@functools.lru_cache(maxsize=None)
def _make_gather(V, D, N):
    info = plsc.get_sparse_core_info()
    NC, NS = info.num_cores, info.num_subcores
    NW = NC * NS
    assert N % (NW * _CHUNK) == 0
    b_per_w = N // NW
    n_chunks = b_per_w // _CHUNK

    mesh = plsc.VectorSubcoreMesh(core_axis_name="c", subcore_axis_name="s")

    @functools.partial(
        pl.kernel,
        mesh=mesh,
        compiler_params=pltpu.CompilerParams(use_tc_tiling_on_sc=False),
        out_type=jax.ShapeDtypeStruct((N, D), jnp.float32),
        scratch_types=[
            pltpu.VMEM((n_chunks, _CHUNK), jnp.int32),
            pltpu.VMEM((2, _CHUNK, D), jnp.float32),
            pltpu.SemaphoreType.DMA,
            pltpu.SemaphoreType.DMA,
            pltpu.SemaphoreType.DMA,
        ],
    )
    def k(idx_hbm, table_hbm, out_hbm, idx_v, rows_v, gsem0, gsem1, osem):
        wid = lax.axis_index("s") * NC + lax.axis_index("c")
        base = wid * b_per_w
        pltpu.sync_copy(idx_hbm.at[wid], idx_v)
        gsems = (gsem0, gsem1)

        pltpu.async_copy(table_hbm.at[idx_v.at[0]], rows_v.at[0], gsem0)

        def body(p, carry):
            for b in range(2):
                j = 2 * p + b
                nxt = j + 1

                @pl.when(nxt < n_chunks)
                def _():
                    @pl.when(nxt >= 2)
                    def _():
                        pltpu.make_async_copy(
                            rows_v.at[1 - b],
                            out_hbm.at[pl.ds(base, _CHUNK)],
                            osem,
                        ).wait()

                    pltpu.async_copy(
                        table_hbm.at[idx_v.at[nxt]], rows_v.at[1 - b], gsems[1 - b]
                    )

                pltpu.make_async_copy(
                    table_hbm.at[idx_v.at[j]], rows_v.at[b], gsems[b]
                ).wait()
                pltpu.async_copy(
                    rows_v.at[b], out_hbm.at[pl.ds(base + j * _CHUNK, _CHUNK)], osem
                )
            return carry

        lax.fori_loop(0, n_chunks // 2, body, 0)
        pltpu.make_async_copy(
            rows_v.at[0], out_hbm.at[pl.ds(base, _CHUNK)], osem
        ).wait()
        pltpu.make_async_copy(
            rows_v.at[1], out_hbm.at[pl.ds(base, _CHUNK)], osem
        ).wait()

    def run(idx2d, table):
        return k(idx2d, table)

    return run


def kernel(tokens, table):
    B, L = tokens.shape
    V, D = table.shape
    N = B * L
    info = plsc.get_sparse_core_info()
    NW = info.num_cores * info.num_subcores
    idx2d = tokens.astype(jnp.int32).reshape(NW, N // (NW * _CHUNK), _CHUNK)
    out = _make_gather(V, D, N)(idx2d, table)
    return out.reshape(B, L, D)


# speedup vs baseline: 1.1131x; 1.0169x over previous
import functools

import jax
import jax.numpy as jnp
from jax import lax
from jax.experimental import pallas as pl
from jax.experimental.pallas import tpu as pltpu
from jax.experimental.pallas import tpu_sc as plsc

_CHUNK = 128
_NBUF = 8
_DEPTH = _NBUF - 2


@functools.lru_cache(maxsize=None)
def _make_gather(V, D, N):
    info = plsc.get_sparse_core_info()
    NC, NS = info.num_cores, info.num_subcores
    NW = NC * NS
    assert N % (NW * _CHUNK) == 0
    b_per_w = N // NW
    n_chunks = b_per_w // _CHUNK

    mesh = plsc.VectorSubcoreMesh(core_axis_name="c", subcore_axis_name="s")

    @functools.partial(
        pl.kernel,
        mesh=mesh,
        compiler_params=pltpu.CompilerParams(use_tc_tiling_on_sc=False),
        out_type=jax.ShapeDtypeStruct((N, D), jnp.float32),
        scratch_types=[
            pltpu.VMEM((n_chunks, _CHUNK), jnp.int32),
            pltpu.VMEM((_NBUF, _CHUNK, D), jnp.float32),
        ]
        + [pltpu.SemaphoreType.DMA] * (_NBUF + 1),
    )
    def k(idx_hbm, table_hbm, out_hbm, idx_v, rows_v, *sems):
        gsems, osem = sems[:_NBUF], sems[_NBUF]
        wid = lax.axis_index("s") * NC + lax.axis_index("c")
        base = wid * b_per_w
        pltpu.sync_copy(idx_hbm.at[wid], idx_v)

        for c in range(_DEPTH):
            pltpu.async_copy(table_hbm.at[idx_v.at[c]], rows_v.at[c], gsems[c])

        def body(p, carry):
            for b in range(_NBUF):
                j = p * _NBUF + b

                @pl.when(j >= 2)
                def _():
                    pltpu.make_async_copy(
                        rows_v.at[b], out_hbm.at[pl.ds(base, _CHUNK)], osem
                    ).wait()

                nb = (b + _DEPTH) % _NBUF

                @pl.when(j + _DEPTH < n_chunks)
                def _():
                    pltpu.async_copy(
                        table_hbm.at[idx_v.at[j + _DEPTH]], rows_v.at[nb], gsems[nb]
                    )

                pltpu.make_async_copy(
                    table_hbm.at[idx_v.at[j]], rows_v.at[b], gsems[b]
                ).wait()
                pltpu.async_copy(
                    rows_v.at[b], out_hbm.at[pl.ds(base + j * _CHUNK, _CHUNK)], osem
                )
            return carry

        lax.fori_loop(0, n_chunks // _NBUF, body, 0)
        for b in range(2):
            pltpu.make_async_copy(
                rows_v.at[b], out_hbm.at[pl.ds(base, _CHUNK)], osem
            ).wait()

    def run(idx2d, table):
        return k(idx2d, table)

    return run


def kernel(tokens, table):
    B, L = tokens.shape
    V, D = table.shape
    N = B * L
    info = plsc.get_sparse_core_info()
    NW = info.num_cores * info.num_subcores
    idx2d = tokens.astype(jnp.int32).reshape(NW, N // (NW * _CHUNK), _CHUNK)
    out = _make_gather(V, D, N)(idx2d, table)
    return out.reshape(B, L, D)
